# baseline (speedup 1.0000x reference)
import jax
import jax.numpy as jnp
from jax import lax
from jax.experimental import pallas as pl
from jax.experimental.pallas import tpu as pltpu
from jax.experimental.pallas import tpu_sc as plsc

N_NODES = 100000
FEAT = 128
B = 50000
S = 10
SLOTS = 12
LG = FEAT // 16

NC = 2
NS = 16
NW = NC * NS

BH = 25000
B_PAD = 26624
B_PER_W = B_PAD // NW
C = 32
NCHUNK = B_PER_W // C
GROWS = SLOTS * C
NSTREAM = GROWS // 128


def _sc_body(idx_hbm, feat_hbm, self_hbm, nsum_hbm,
             idx_c0, idx_c1, gbuf0, gbuf1, nout,
             gsem0, gsem1, osem0, osem1, isem0, isem1):
    cc = lax.axis_index("c")
    s = lax.axis_index("s")
    wid = s * NC + cc

    def fire_gathers(idx_c, gbuf, gsem):
        for j in range(NSTREAM):
            pltpu.async_copy(feat_hbm.at[idx_c.at[j]],
                             gbuf.at[pl.ds(j * 128, 128)], gsem)

    def drain_reduce(ci, idx_c, gbuf, gsem, p, osem, isem, first):
        base = wid * B_PER_W + ci * C
        for j in range(NSTREAM):
            pltpu.make_async_copy(
                feat_hbm.at[pl.ds(0, 128)],
                gbuf.at[pl.ds(j * 128, 128)], gsem).wait()
        @pl.when(ci + 2 < NCHUNK)
        def _():
            pltpu.async_copy(idx_hbm.at[wid, ci + 2], idx_c, isem)
        @pl.when(jnp.logical_not(first))
        def _():
            pltpu.make_async_copy(
                feat_hbm.at[pl.ds(0, C)], gbuf.at[pl.ds(0, C)], osem).wait()
        pltpu.async_copy(gbuf.at[pl.ds(0, C)], self_hbm.at[pl.ds(base, C)],
                         osem)

        def seed_body(i, carry):
            for g in range(LG):
                acc = gbuf[C + i, pl.ds(16 * g, 16)]
                for t in range(2, SLOTS - 1):
                    acc = acc + gbuf[t * C + i, pl.ds(16 * g, 16)]
                nout[p, i, pl.ds(16 * g, 16)] = acc
            return carry

        lax.fori_loop(0, C, seed_body, 0)

        pltpu.make_async_copy(
            feat_hbm.at[pl.ds(0, C)], gbuf.at[pl.ds(0, C)], osem).wait()
        pltpu.async_copy(nout.at[p], nsum_hbm.at[pl.ds(base, C)], osem)

        @pl.when(ci + 2 < NCHUNK)
        def _():
            pltpu.make_async_copy(
                idx_hbm.at[wid, 0], idx_c, isem).wait()
            fire_gathers(idx_c, gbuf, gsem)

    pltpu.sync_copy(idx_hbm.at[wid, 0], idx_c0)
    pltpu.sync_copy(idx_hbm.at[wid, 1], idx_c1)
    fire_gathers(idx_c0, gbuf0, gsem0)
    fire_gathers(idx_c1, gbuf1, gsem1)

    def body(k, carry):
        drain_reduce(2 * k, idx_c0, gbuf0, gsem0, 0, osem0, isem0, k == 0)
        drain_reduce(2 * k + 1, idx_c1, gbuf1, gsem1, 1, osem1, isem1,
                     k == 0)
        return carry

    lax.fori_loop(0, NCHUNK // 2, body, 0)

    pltpu.make_async_copy(
        feat_hbm.at[pl.ds(0, C)], gbuf0.at[pl.ds(0, C)], osem0).wait()
    pltpu.make_async_copy(
        feat_hbm.at[pl.ds(0, C)], gbuf1.at[pl.ds(0, C)], osem1).wait()


def _tc_body(self_ref, nsum_ref, ws_ref, wn_ref, w1s_ref, w1n_ref, o_ref):
    xs = self_ref[...]
    xn = nsum_ref[...] * jnp.float32(1.0 / S)
    comb = jnp.tanh(
        jnp.dot(xs, ws_ref[...], preferred_element_type=jnp.float32)
        + jnp.dot(xn, wn_ref[...], preferred_element_type=jnp.float32)
    )
    att = jax.nn.sigmoid(
        jnp.dot(xs, w1s_ref[...], preferred_element_type=jnp.float32)
        + jnp.dot(xn, w1n_ref[...], preferred_element_type=jnp.float32)
    )
    o_ref[...] = att * (comb + jnp.tanh(xn))


_sc_gather = pl.kernel(
    _sc_body,
    out_type=[
        jax.ShapeDtypeStruct((B_PAD, FEAT), jnp.float32),
        jax.ShapeDtypeStruct((B_PAD, FEAT), jnp.float32),
    ],
    mesh=plsc.VectorSubcoreMesh(core_axis_name="c", subcore_axis_name="s"),
    scratch_types=[
        pltpu.VMEM((NSTREAM, 128), jnp.int32),
        pltpu.VMEM((NSTREAM, 128), jnp.int32),
        pltpu.VMEM((GROWS, FEAT), jnp.float32),
        pltpu.VMEM((GROWS, FEAT), jnp.float32),
        pltpu.VMEM((2, C, FEAT), jnp.float32),
        pltpu.SemaphoreType.DMA,
        pltpu.SemaphoreType.DMA,
        pltpu.SemaphoreType.DMA,
        pltpu.SemaphoreType.DMA,
        pltpu.SemaphoreType.DMA,
        pltpu.SemaphoreType.DMA,
    ],
)

_TC_R = 1000


def _make_idx(nodes_h, neigh_h, salt):
    pad = B_PAD - BH
    nodes_p = jnp.concatenate(
        [nodes_h, (salt + jnp.arange(pad, dtype=jnp.int32)) % N_NODES])
    neigh_p = jnp.concatenate(
        [neigh_h,
         ((salt + jnp.arange(pad * S, dtype=jnp.int32))
          % N_NODES).reshape(pad, S)])
    self_i = nodes_p.reshape(NW, NCHUNK, 1, C)
    neigh_i = neigh_p.reshape(NW, NCHUNK, C, S).transpose(0, 1, 3, 2)
    dummy_i = ((salt + jnp.arange(NW * NCHUNK * C, dtype=jnp.int32))
               % N_NODES).reshape(NW, NCHUNK, 1, C)
    idx_all = jnp.concatenate([self_i, neigh_i, dummy_i], axis=2)
    return idx_all.reshape(NW, NCHUNK, NSTREAM, 128)


def _tc_half(self_f, nsum_f, ws, wn, w1s, w1n):
    wspec = pl.BlockSpec((FEAT, FEAT), lambda i: (0, 0))
    return pl.pallas_call(
        _tc_body,
        grid=(BH // _TC_R,),
        in_specs=[
            pl.BlockSpec((_TC_R, FEAT), lambda i: (i, 0)),
            pl.BlockSpec((_TC_R, FEAT), lambda i: (i, 0)),
            wspec, wspec, wspec, wspec,
        ],
        out_specs=pl.BlockSpec((_TC_R, FEAT), lambda i: (i, 0)),
        out_shape=jax.ShapeDtypeStruct((BH, FEAT), jnp.float32),
    )(self_f, nsum_f, ws, wn, w1s, w1n)


@jax.jit
def _run(nodes, neigh_idx, features, weight, weight1):
    ws = weight[:, :FEAT].T
    wn = weight[:, FEAT:].T
    w1s = weight1[:, :FEAT].T
    w1n = weight1[:, FEAT:].T

    outs = []
    for h in range(2):
        idx_h = _make_idx(nodes[h * BH:(h + 1) * BH],
                          neigh_idx[h * BH:(h + 1) * BH], 83 * h)
        self_f, nsum_f = _sc_gather(idx_h, features)
        outs.append(_tc_half(self_f, nsum_f, ws, wn, w1s, w1n))
    return jnp.concatenate(outs, axis=0)


def kernel(nodes, neigh_idx, features, weight, weight1):
    return _run(nodes, neigh_idx, features, weight, weight1)


# speedup vs baseline: 1.1232x; 1.1232x over previous
import jax
import jax.numpy as jnp
from jax import lax
from jax.experimental import pallas as pl
from jax.experimental.pallas import tpu as pltpu
from jax.experimental.pallas import tpu_sc as plsc

N_NODES = 100000
FEAT = 128
B = 50000
S = 10
SLOTS = 12
LG = FEAT // 16

NC = 2
NS = 16
NW = NC * NS

B_PAD = 51200
B_PER_W = B_PAD // NW
C = 32
NCHUNK = B_PER_W // C
GROWS = SLOTS * C
NSTREAM = GROWS // 128


def _sc_body(idx_hbm, feat_hbm, self_hbm, nsum_hbm,
             idx_c0, idx_c1, gbuf0, gbuf1, nout,
             gsem0, gsem1, osem0, osem1, isem0, isem1):
    cc = lax.axis_index("c")
    s = lax.axis_index("s")
    wid = s * NC + cc

    def fire_gathers(idx_c, gbuf, gsem):
        for j in range(NSTREAM):
            pltpu.async_copy(feat_hbm.at[idx_c.at[j]],
                             gbuf.at[pl.ds(j * 128, 128)], gsem)

    def drain_reduce(ci, idx_c, gbuf, gsem, p, osem, isem, first):
        base = wid * B_PER_W + ci * C
        for j in range(NSTREAM):
            pltpu.make_async_copy(
                feat_hbm.at[pl.ds(0, 128)],
                gbuf.at[pl.ds(j * 128, 128)], gsem).wait()
        @pl.when(ci + 2 < NCHUNK)
        def _():
            pltpu.async_copy(idx_hbm.at[wid, ci + 2], idx_c, isem)
        @pl.when(jnp.logical_not(first))
        def _():
            pltpu.make_async_copy(
                feat_hbm.at[pl.ds(0, C)], gbuf.at[pl.ds(0, C)], osem).wait()
        pltpu.async_copy(gbuf.at[pl.ds(0, C)], self_hbm.at[pl.ds(base, C)],
                         osem)

        def seed_body(i, carry):
            for g in range(LG):
                acc = gbuf[C + i, pl.ds(16 * g, 16)]
                for t in range(2, SLOTS - 1):
                    acc = acc + gbuf[t * C + i, pl.ds(16 * g, 16)]
                nout[p, i, pl.ds(16 * g, 16)] = acc
            return carry

        lax.fori_loop(0, C, seed_body, 0)

        pltpu.make_async_copy(
            feat_hbm.at[pl.ds(0, C)], gbuf.at[pl.ds(0, C)], osem).wait()
        pltpu.async_copy(nout.at[p], nsum_hbm.at[pl.ds(base, C)], osem)

        @pl.when(ci + 2 < NCHUNK)
        def _():
            pltpu.make_async_copy(
                idx_hbm.at[wid, 0], idx_c, isem).wait()
            fire_gathers(idx_c, gbuf, gsem)

    pltpu.sync_copy(idx_hbm.at[wid, 0], idx_c0)
    pltpu.sync_copy(idx_hbm.at[wid, 1], idx_c1)
    fire_gathers(idx_c0, gbuf0, gsem0)
    fire_gathers(idx_c1, gbuf1, gsem1)

    def body(k, carry):
        drain_reduce(2 * k, idx_c0, gbuf0, gsem0, 0, osem0, isem0, k == 0)
        drain_reduce(2 * k + 1, idx_c1, gbuf1, gsem1, 1, osem1, isem1,
                     k == 0)
        return carry

    lax.fori_loop(0, NCHUNK // 2, body, 0)

    pltpu.make_async_copy(
        feat_hbm.at[pl.ds(0, C)], gbuf0.at[pl.ds(0, C)], osem0).wait()
    pltpu.make_async_copy(
        feat_hbm.at[pl.ds(0, C)], gbuf1.at[pl.ds(0, C)], osem1).wait()


def _tc_body(self_ref, nsum_ref, ws_ref, wn_ref, w1s_ref, w1n_ref, o_ref):
    xs = self_ref[...]
    xn = nsum_ref[...] * jnp.float32(1.0 / S)
    comb = jnp.tanh(
        jnp.dot(xs, ws_ref[...], preferred_element_type=jnp.float32)
        + jnp.dot(xn, wn_ref[...], preferred_element_type=jnp.float32)
    )
    att = jax.nn.sigmoid(
        jnp.dot(xs, w1s_ref[...], preferred_element_type=jnp.float32)
        + jnp.dot(xn, w1n_ref[...], preferred_element_type=jnp.float32)
    )
    o_ref[...] = att * (comb + jnp.tanh(xn))


_sc_gather = pl.kernel(
    _sc_body,
    out_type=[
        jax.ShapeDtypeStruct((B_PAD, FEAT), jnp.float32),
        jax.ShapeDtypeStruct((B_PAD, FEAT), jnp.float32),
    ],
    mesh=plsc.VectorSubcoreMesh(core_axis_name="c", subcore_axis_name="s"),
    scratch_types=[
        pltpu.VMEM((NSTREAM, 128), jnp.int32),
        pltpu.VMEM((NSTREAM, 128), jnp.int32),
        pltpu.VMEM((GROWS, FEAT), jnp.float32),
        pltpu.VMEM((GROWS, FEAT), jnp.float32),
        pltpu.VMEM((2, C, FEAT), jnp.float32),
        pltpu.SemaphoreType.DMA,
        pltpu.SemaphoreType.DMA,
        pltpu.SemaphoreType.DMA,
        pltpu.SemaphoreType.DMA,
        pltpu.SemaphoreType.DMA,
        pltpu.SemaphoreType.DMA,
    ],
)

_TC_R = 2000


@jax.jit
def _run(nodes, neigh_idx, features, weight, weight1):
    pad = B_PAD - B
    nodes_p = jnp.concatenate(
        [nodes, jnp.arange(pad, dtype=jnp.int32) % N_NODES])
    neigh_p = jnp.concatenate(
        [neigh_idx,
         (jnp.arange(pad * S, dtype=jnp.int32) % N_NODES).reshape(pad, S)])
    self_i = nodes_p.reshape(NW, NCHUNK, 1, C)
    neigh_i = neigh_p.reshape(NW, NCHUNK, C, S).transpose(0, 1, 3, 2)
    dummy_i = (jnp.arange(NW * NCHUNK * C, dtype=jnp.int32)
               % N_NODES).reshape(NW, NCHUNK, 1, C)
    idx_all = jnp.concatenate([self_i, neigh_i, dummy_i], axis=2)
    idx_all = idx_all.reshape(NW, NCHUNK, NSTREAM, 128)

    self_f, nsum_f = _sc_gather(idx_all, features)

    ws = weight[:, :FEAT].T
    wn = weight[:, FEAT:].T
    w1s = weight1[:, :FEAT].T
    w1n = weight1[:, FEAT:].T

    wspec = pl.BlockSpec((FEAT, FEAT), lambda i: (0, 0))
    out = pl.pallas_call(
        _tc_body,
        grid=(B // _TC_R,),
        in_specs=[
            pl.BlockSpec((_TC_R, FEAT), lambda i: (i, 0)),
            pl.BlockSpec((_TC_R, FEAT), lambda i: (i, 0)),
            wspec, wspec, wspec, wspec,
        ],
        out_specs=pl.BlockSpec((_TC_R, FEAT), lambda i: (i, 0)),
        out_shape=jax.ShapeDtypeStruct((B, FEAT), jnp.float32),
    )(self_f, nsum_f, ws, wn, w1s, w1n)
    return out


def kernel(nodes, neigh_idx, features, weight, weight1):
    return _run(nodes, neigh_idx, features, weight, weight1)
